# Initial kernel scaffold; baseline (speedup 1.0000x reference)
import jax
import jax.numpy as jnp
from jax.experimental import pallas as pl


def kernel(state, rm_state, W0, b0, W1, b1, W2, b2, W3, b3, W4, b4, W5, b5):
    raise NotImplementedError("write your pallas kernel here")


# speedup vs baseline: 1.1583x; 1.1583x over previous
import functools

import jax
import jax.numpy as jnp
from jax import lax
from jax.experimental import pallas as pl
from jax.experimental.pallas import tpu as pltpu
from jax.experimental.pallas import tpu_sc as plsc

_NC = 2
_NS = 16
_NW = _NC * _NS
_CHUNK = 128


def _pass_a_body(x_ref, w_ref, b_ref, o_ref):
    xb = x_ref[...].astype(jnp.bfloat16)
    acc = jnp.dot(xb, w_ref[...], preferred_element_type=jnp.float32)
    o_ref[...] = jnp.maximum(acc + b_ref[...], 0.0)


def _pass_a(state, w0all, b0all, block_rows=512):
    b, d = state.shape
    eh = w0all.shape[1]
    return pl.pallas_call(
        _pass_a_body,
        grid=(b // block_rows,),
        in_specs=[
            pl.BlockSpec((block_rows, d), lambda i: (i, 0)),
            pl.BlockSpec((d, eh), lambda i: (0, 0)),
            pl.BlockSpec((1, eh), lambda i: (0, 0)),
        ],
        out_specs=pl.BlockSpec((block_rows, eh), lambda i: (i, 0)),
        out_shape=jax.ShapeDtypeStruct((b, eh), jnp.float32),
    )(state, w0all, b0all)


def _sc_dispatch(h0_rows, idx_src, idx_dst, p_rows):
    nw, k, c = idx_src.shape
    per_w = k * c
    width = h0_rows.shape[1]
    mesh = plsc.VectorSubcoreMesh(core_axis_name="c", subcore_axis_name="s")

    @functools.partial(
        pl.kernel,
        mesh=mesh,
        out_type=jax.ShapeDtypeStruct((p_rows, width), jnp.float32),
        compiler_params=pltpu.CompilerParams(use_tc_tiling_on_sc=False),
        scratch_types=[
            pltpu.VMEM((k, c), jnp.int32),
            pltpu.VMEM((k, c), jnp.int32),
            pltpu.VMEM((per_w, width), jnp.float32),
            pltpu.SemaphoreType.DMA,
        ],
    )
    def kern(h0_hbm, isrc_hbm, idst_hbm, xpad_hbm, isrc_v, idst_v, rows_v, sem):
        wid = lax.axis_index("s") * _NC + lax.axis_index("c")
        pltpu.sync_copy(isrc_hbm.at[wid], isrc_v)
        pltpu.sync_copy(idst_hbm.at[wid], idst_v)
        gathers = []
        for j in range(k):
            gathers.append(pltpu.async_copy(
                h0_hbm.at[isrc_v.at[j]],
                rows_v.at[pl.ds(j * c, c)], sem))
        scatters = []
        for j in range(k):
            gathers[j].wait()
            scatters.append(pltpu.async_copy(
                rows_v.at[pl.ds(j * c, c)],
                xpad_hbm.at[idst_v.at[j]], sem))
        for s in scatters:
            s.wait()

    return kern(h0_rows, idx_src, idx_dst)


def _sc_collect(y_pad, idx, b_rows):
    nw, k, c = idx.shape
    per_w = k * c
    width = y_pad.shape[1]
    mesh = plsc.VectorSubcoreMesh(core_axis_name="c", subcore_axis_name="s")

    @functools.partial(
        pl.kernel,
        mesh=mesh,
        out_type=jax.ShapeDtypeStruct((b_rows, width), jnp.float32),
        compiler_params=pltpu.CompilerParams(use_tc_tiling_on_sc=False),
        scratch_types=[
            pltpu.VMEM((k, c), jnp.int32),
            pltpu.VMEM((per_w, width), jnp.float32),
            pltpu.SemaphoreType.DMA,
        ],
    )
    def kern(ypad_hbm, idx_hbm, out_hbm, idx_v, rows_v, sem):
        wid = lax.axis_index("s") * _NC + lax.axis_index("c")
        pltpu.sync_copy(idx_hbm.at[wid], idx_v)
        gathers = []
        for j in range(k):
            gathers.append(pltpu.async_copy(
                ypad_hbm.at[idx_v.at[j]],
                rows_v.at[pl.ds(j * c, c)], sem))
        for g in gathers:
            g.wait()
        pltpu.sync_copy(rows_v, out_hbm.at[pl.ds(wid * per_w, per_w)])

    return kern(y_pad, idx)


def _pass_b_body(se_ref, x_ref, w1_ref, w2_ref, w3_ref, w4_ref, w5_ref,
                 bt_ref, o_ref):
    h = x_ref[...].astype(jnp.bfloat16)
    for l, w_ref in enumerate((w1_ref, w2_ref, w3_ref, w4_ref)):
        acc = jnp.dot(h, w_ref[0], preferred_element_type=jnp.float32)
        h = jnp.maximum(acc + bt_ref[0, l, :], 0.0).astype(jnp.bfloat16)
    o_ref[...] = (jnp.dot(h, w5_ref[0], preferred_element_type=jnp.float32)
                  + bt_ref[0, 4, :])


def _pass_b(tile_expert, x_pad, ws_bf, btile, tile_rows, n_tiles, h, a):
    w_spec = pl.BlockSpec((1, h, h), lambda t, se: (se[t], 0, 0))
    grid_spec = pltpu.PrefetchScalarGridSpec(
        num_scalar_prefetch=1,
        grid=(n_tiles,),
        in_specs=[
            pl.BlockSpec((tile_rows, h), lambda t, se: (t, 0)),
            w_spec, w_spec, w_spec, w_spec,
            pl.BlockSpec((1, h, a), lambda t, se: (se[t], 0, 0)),
            pl.BlockSpec((1, 8, a), lambda t, se: (t, 0, 0)),
        ],
        out_specs=pl.BlockSpec((tile_rows, a), lambda t, se: (t, 0)),
    )
    return pl.pallas_call(
        _pass_b_body,
        grid_spec=grid_spec,
        out_shape=jax.ShapeDtypeStruct((n_tiles * tile_rows, a), jnp.float32),
    )(tile_expert, x_pad, *ws_bf, btile)


def kernel(state, rm_state, W0, b0, W1, b1, W2, b2, W3, b3, W4, b4, W5, b5):
    B, D = state.shape
    E, _, H = W0.shape
    A = W5.shape[2]
    T = 512
    NT = B // T + E
    P = NT * T

    e = rm_state.astype(jnp.int32)
    oh = (e[:, None] == jnp.arange(E, dtype=jnp.int32)[None, :]).astype(jnp.int32)
    cs = jnp.cumsum(oh, axis=0)
    cnt = cs[-1]
    occ = jnp.sum((cs - oh) * oh, axis=1)
    tiles_e = (cnt + T - 1) // T
    tile_start = jnp.concatenate(
        [jnp.zeros((1,), jnp.int32), jnp.cumsum(tiles_e)[:-1].astype(jnp.int32)])
    row_start = tile_start * T
    p = jnp.sum(oh * row_start[None, :], axis=1) + occ
    idx_src = (jnp.arange(B, dtype=jnp.int32) * E + e).reshape(_NW, -1, _CHUNK)
    idx_dst = p.reshape(_NW, -1, _CHUNK)
    tile_expert = (jnp.sum(
        (jnp.arange(NT, dtype=jnp.int32)[:, None] >= tile_start[None, :])
        .astype(jnp.int32), axis=1) - 1)

    w0all = W0.transpose(1, 0, 2).reshape(D, E * H).astype(jnp.bfloat16)
    b0all = b0.reshape(1, E * H)
    ws_bf = tuple(w.astype(jnp.bfloat16) for w in (W1, W2, W3, W4, W5))
    bstack = jnp.stack((b1, b2, b3, b4, b5), axis=1)
    bstack = jnp.pad(bstack, ((0, 0), (0, 3), (0, 0)))
    oh_t = (tile_expert[:, None] == jnp.arange(E, dtype=jnp.int32)[None, :])
    btile = jnp.einsum('te,ela->tla', oh_t.astype(jnp.float32),
                       bstack)

    h0 = _pass_a(state, w0all, b0all)
    h0_rows = h0.reshape(B * E, H)
    x_pad = _sc_dispatch(h0_rows, idx_src, idx_dst, P)
    y_pad = _pass_b(tile_expert, x_pad, ws_bf, btile, T, NT, H, A)
    actions = _sc_collect(y_pad, idx_dst, B)
    return actions
